# baseline (speedup 1.0000x reference)
import jax
import jax.numpy as jnp
from jax import lax
from jax.experimental import pallas as pl
from jax.experimental.pallas import tpu as pltpu
from jax.experimental.pallas import tpu_sc as plsc

NUM_CORES = 2
NUM_SUBCORES = 16
LANES = 16
NW = NUM_CORES * NUM_SUBCORES

BATCH = 16384
HIDDEN = 128
ROWS_PER_W = BATCH // NW
CHUNK = 64
NCHUNK = ROWS_PER_W // CHUNK
NGROUP = CHUNK // LANES
NV = HIDDEN // LANES


def _tree_sum(vals):
    while len(vals) > 1:
        vals = [vals[i] + vals[i + 1] for i in range(0, len(vals) - 1, 2)] \
            + ([vals[-1]] if len(vals) % 2 else [])
    return vals[0]


def _sc_body(users_hbm, items_hbm, vf_hbm, tf_hbm,
             ug_hbm, ig_hbm, ubeta_hbm, ibeta_hbm, tv_hbm, tt_hbm,
             out_hbm,
             idxu, idxi, idxud, idxid, ug, ig, tv, tt, vf, tf, ub, ib,
             racc, score, gsems, osems):
    wid = lax.axis_index("s") * NUM_CORES + lax.axis_index("c")
    base = wid * ROWS_PER_W

    pltpu.sync_copy(users_hbm.at[pl.ds(base, ROWS_PER_W)], idxu)
    pltpu.sync_copy(items_hbm.at[pl.ds(base, ROWS_PER_W)], idxi)
    for t in range(ROWS_PER_W // LANES):
        d = pl.ds(t * LANES, LANES)
        idxud[d] = jax.lax.shift_right_logical(idxu[d], 4)
        idxid[d] = jax.lax.shift_right_logical(idxi[d], 4)

    def fire_part(c, k):
        s = c % 2
        row0 = base + c * CHUNK
        loc = pl.ds(c * CHUNK, CHUNK)
        pairs = [
            lambda: [
                pltpu.async_copy(ug_hbm.at[idxu.at[loc]], ug[s], gsems[s]),
                pltpu.async_copy(ig_hbm.at[idxi.at[loc]], ig[s], gsems[s]),
            ],
            lambda: [
                pltpu.async_copy(tv_hbm.at[idxu.at[loc]], tv[s], gsems[s]),
                pltpu.async_copy(tt_hbm.at[idxu.at[loc]], tt[s], gsems[s]),
            ],
            lambda: [
                pltpu.async_copy(vf_hbm.at[pl.ds(row0, CHUNK), :], vf[s],
                                 gsems[s]),
                pltpu.async_copy(tf_hbm.at[pl.ds(row0, CHUNK), :], tf[s],
                                 gsems[s]),
            ],
            lambda: [
                pltpu.async_copy(ubeta_hbm.at[idxud.at[loc]], ub[s],
                                 gsems[s]),
                pltpu.async_copy(ibeta_hbm.at[idxid.at[loc]], ib[s],
                                 gsems[s]),
            ],
        ]
        return pairs[k]()

    lane = lax.iota(jnp.int32, LANES)

    def compute(c, issue_next):
        s = c % 2
        nxt = []

        for g in range(NGROUP):
            r0 = g * LANES

            def row_body(i, _, r0=r0):
                r = r0 + i
                prods = []
                for v in range(NV):
                    d = pl.ds(v * LANES, LANES)
                    prods.append(ug[s][r, d] * ig[s][r, d])
                    prods.append(tv[s][r, d] * vf[s][r, d])
                    prods.append(tt[s][r, d] * tf[s][r, d])
                plsc.store_scatter(racc, [lane * CHUNK + r], _tree_sum(prods))
                return 0

            lax.fori_loop(0, LANES, row_body, 0, unroll=2)
            if issue_next:
                nxt += fire_part(c + 1, g)

        for g in range(NGROUP):
            r0 = g * LANES
            ridx = r0 + lane
            ulo = jnp.bitwise_and(idxu[pl.ds(c * CHUNK + r0, LANES)], 15)
            ilo = jnp.bitwise_and(idxi[pl.ds(c * CHUNK + r0, LANES)], 15)
            sv = (plsc.load_gather(ub[s], [ridx, ulo])
                  + plsc.load_gather(ib[s], [ridx, ilo]))
            cols = [racc[pl.ds(l * CHUNK + r0, LANES)] for l in range(LANES)]
            score[s][pl.ds(r0, LANES)] = sv + _tree_sum(cols)

        ofut = pltpu.async_copy(
            score[s], out_hbm.at[pl.ds(base + c * CHUNK, CHUNK)], osems[s])
        return ofut, nxt

    futs = {0: [f for k in range(NGROUP) for f in fire_part(0, k)]}
    ofuts = {}
    for c in range(NCHUNK):
        for f in futs.pop(c):
            f.wait()
        if c >= 2:
            ofuts.pop(c - 2).wait()
        ofuts[c], nxt = compute(c, c + 1 < NCHUNK)
        if nxt:
            futs[c + 1] = nxt
    for c in sorted(ofuts):
        ofuts[c].wait()


def kernel(users, items, visual_features, textural_features,
           user_gama, item_gama, user_beta, item_beta,
           theta_user_visual, theta_user_text):
    mesh = plsc.VectorSubcoreMesh(core_axis_name="c", subcore_axis_name="s")
    scratch = (
        pltpu.VMEM((ROWS_PER_W,), jnp.int32),
        pltpu.VMEM((ROWS_PER_W,), jnp.int32),
        pltpu.VMEM((ROWS_PER_W,), jnp.int32),
        pltpu.VMEM((ROWS_PER_W,), jnp.int32),
        [pltpu.VMEM((CHUNK, HIDDEN), jnp.float32) for _ in range(2)],
        [pltpu.VMEM((CHUNK, HIDDEN), jnp.float32) for _ in range(2)],
        [pltpu.VMEM((CHUNK, HIDDEN), jnp.float32) for _ in range(2)],
        [pltpu.VMEM((CHUNK, HIDDEN), jnp.float32) for _ in range(2)],
        [pltpu.VMEM((CHUNK, HIDDEN), jnp.float32) for _ in range(2)],
        [pltpu.VMEM((CHUNK, HIDDEN), jnp.float32) for _ in range(2)],
        [pltpu.VMEM((CHUNK, LANES), jnp.float32) for _ in range(2)],
        [pltpu.VMEM((CHUNK, LANES), jnp.float32) for _ in range(2)],
        pltpu.VMEM((LANES * CHUNK,), jnp.float32),
        [pltpu.VMEM((CHUNK,), jnp.float32) for _ in range(2)],
        [pltpu.SemaphoreType.DMA for _ in range(2)],
        [pltpu.SemaphoreType.DMA for _ in range(2)],
    )
    run = pl.kernel(
        _sc_body,
        out_type=jax.ShapeDtypeStruct((BATCH,), jnp.float32),
        mesh=mesh,
        scratch_types=scratch,
        compiler_params=pltpu.CompilerParams(
            needs_layout_passes=False, use_tc_tiling_on_sc=False),
    )
    return run(users.astype(jnp.int32), items.astype(jnp.int32),
               visual_features, textural_features,
               user_gama, item_gama,
               user_beta[:, 0].reshape(-1, LANES),
               item_beta[:, 0].reshape(-1, LANES),
               theta_user_visual, theta_user_text)


# speedup vs baseline: 1.0677x; 1.0677x over previous
import jax
import jax.numpy as jnp
from jax import lax
from jax.experimental import pallas as pl
from jax.experimental.pallas import tpu as pltpu
from jax.experimental.pallas import tpu_sc as plsc

NUM_CORES = 2
NUM_SUBCORES = 16
LANES = 16
NW = NUM_CORES * NUM_SUBCORES

BATCH = 16384
HIDDEN = 128
ROWS_PER_W = BATCH // NW
CHUNK = 32
NBUF = 3
NCHUNK = ROWS_PER_W // CHUNK
NGROUP = CHUNK // LANES
NV = HIDDEN // LANES


def _tree_sum(vals):
    while len(vals) > 1:
        vals = [vals[i] + vals[i + 1] for i in range(0, len(vals) - 1, 2)] \
            + ([vals[-1]] if len(vals) % 2 else [])
    return vals[0]


def _sc_body(users_hbm, items_hbm, vf_hbm, tf_hbm,
             ug_hbm, ig_hbm, ubeta_hbm, ibeta_hbm, tv_hbm, tt_hbm,
             out_hbm,
             idxu, idxi, idxud, idxid, ug, ig, tv, tt, vf, tf, ub, ib,
             racc, score, gsems, osems):
    wid = lax.axis_index("s") * NUM_CORES + lax.axis_index("c")
    base = wid * ROWS_PER_W

    pltpu.sync_copy(users_hbm.at[pl.ds(base, ROWS_PER_W)], idxu)
    pltpu.sync_copy(items_hbm.at[pl.ds(base, ROWS_PER_W)], idxi)
    for t in range(ROWS_PER_W // LANES):
        d = pl.ds(t * LANES, LANES)
        idxud[d] = jax.lax.shift_right_logical(idxu[d], 4)
        idxid[d] = jax.lax.shift_right_logical(idxi[d], 4)

    def fire(c):
        s = c % NBUF
        row0 = base + c * CHUNK
        loc = pl.ds(c * CHUNK, CHUNK)
        return [
            pltpu.async_copy(ug_hbm.at[idxu.at[loc]], ug[s], gsems[s]),
            pltpu.async_copy(ig_hbm.at[idxi.at[loc]], ig[s], gsems[s]),
            pltpu.async_copy(tv_hbm.at[idxu.at[loc]], tv[s], gsems[s]),
            pltpu.async_copy(tt_hbm.at[idxu.at[loc]], tt[s], gsems[s]),
            pltpu.async_copy(ubeta_hbm.at[idxud.at[loc]], ub[s], gsems[s]),
            pltpu.async_copy(ibeta_hbm.at[idxid.at[loc]], ib[s], gsems[s]),
            pltpu.async_copy(vf_hbm.at[pl.ds(row0, CHUNK), :], vf[s], gsems[s]),
            pltpu.async_copy(tf_hbm.at[pl.ds(row0, CHUNK), :], tf[s], gsems[s]),
        ]

    lane = lax.iota(jnp.int32, LANES)

    def compute(c):
        s = c % NBUF

        def row_body(r, _):
            prods = []
            for v in range(NV):
                d = pl.ds(v * LANES, LANES)
                prods.append(ug[s][r, d] * ig[s][r, d])
                prods.append(tv[s][r, d] * vf[s][r, d])
                prods.append(tt[s][r, d] * tf[s][r, d])
            plsc.store_scatter(racc, [lane * CHUNK + r], _tree_sum(prods))
            return 0

        lax.fori_loop(0, CHUNK, row_body, 0, unroll=4)

        for g in range(NGROUP):
            r0 = g * LANES
            ridx = r0 + lane
            ulo = jnp.bitwise_and(idxu[pl.ds(c * CHUNK + r0, LANES)], 15)
            ilo = jnp.bitwise_and(idxi[pl.ds(c * CHUNK + r0, LANES)], 15)
            sv = (plsc.load_gather(ub[s], [ridx, ulo])
                  + plsc.load_gather(ib[s], [ridx, ilo]))
            cols = [racc[pl.ds(l * CHUNK + r0, LANES)] for l in range(LANES)]
            score[s][pl.ds(r0, LANES)] = sv + _tree_sum(cols)

        return pltpu.async_copy(
            score[s], out_hbm.at[pl.ds(base + c * CHUNK, CHUNK)], osems[s])

    depth = NBUF - 1
    futs = {c: fire(c) for c in range(min(depth, NCHUNK))}
    ofuts = {}
    for c in range(NCHUNK):
        if c + depth < NCHUNK:
            futs[c + depth] = fire(c + depth)
        for f in futs.pop(c):
            f.wait()
        if c >= NBUF:
            ofuts.pop(c - NBUF).wait()
        ofuts[c] = compute(c)
    for c in sorted(ofuts):
        ofuts[c].wait()


def kernel(users, items, visual_features, textural_features,
           user_gama, item_gama, user_beta, item_beta,
           theta_user_visual, theta_user_text):
    mesh = plsc.VectorSubcoreMesh(core_axis_name="c", subcore_axis_name="s")
    scratch = (
        pltpu.VMEM((ROWS_PER_W,), jnp.int32),
        pltpu.VMEM((ROWS_PER_W,), jnp.int32),
        pltpu.VMEM((ROWS_PER_W,), jnp.int32),
        pltpu.VMEM((ROWS_PER_W,), jnp.int32),
        [pltpu.VMEM((CHUNK, HIDDEN), jnp.float32) for _ in range(NBUF)],
        [pltpu.VMEM((CHUNK, HIDDEN), jnp.float32) for _ in range(NBUF)],
        [pltpu.VMEM((CHUNK, HIDDEN), jnp.float32) for _ in range(NBUF)],
        [pltpu.VMEM((CHUNK, HIDDEN), jnp.float32) for _ in range(NBUF)],
        [pltpu.VMEM((CHUNK, HIDDEN), jnp.float32) for _ in range(NBUF)],
        [pltpu.VMEM((CHUNK, HIDDEN), jnp.float32) for _ in range(NBUF)],
        [pltpu.VMEM((CHUNK, LANES), jnp.float32) for _ in range(NBUF)],
        [pltpu.VMEM((CHUNK, LANES), jnp.float32) for _ in range(NBUF)],
        pltpu.VMEM((LANES * CHUNK,), jnp.float32),
        [pltpu.VMEM((CHUNK,), jnp.float32) for _ in range(NBUF)],
        [pltpu.SemaphoreType.DMA for _ in range(NBUF)],
        [pltpu.SemaphoreType.DMA for _ in range(NBUF)],
    )
    run = pl.kernel(
        _sc_body,
        out_type=jax.ShapeDtypeStruct((BATCH,), jnp.float32),
        mesh=mesh,
        scratch_types=scratch,
        compiler_params=pltpu.CompilerParams(
            needs_layout_passes=False, use_tc_tiling_on_sc=False),
    )
    return run(users.astype(jnp.int32), items.astype(jnp.int32),
               visual_features, textural_features,
               user_gama, item_gama,
               user_beta[:, 0].reshape(-1, LANES),
               item_beta[:, 0].reshape(-1, LANES),
               theta_user_visual, theta_user_text)
